# Initial kernel scaffold; baseline (speedup 1.0000x reference)
import jax
import jax.numpy as jnp
from jax.experimental import pallas as pl


def kernel(x_user, x_movie, edge_label_index, W, b):
    raise NotImplementedError("write your pallas kernel here")


# speedup vs baseline: 1.8965x; 1.8965x over previous
import functools

import jax
import jax.numpy as jnp
from jax import lax
from jax.experimental import pallas as pl
from jax.experimental.pallas import tpu as pltpu
from jax.experimental.pallas import tpu_sc as plsc

HIDDEN = 64
N_NODES = 100000
E = 1000000
D = 16

_NC = 2
_NS = 16
_NW = _NC * _NS
_CH = 1000
_NCHUNK = E // _CH
_ROUNDS = _NCHUNK // _NW
_TAIL = _NCHUNK - _ROUNDS * _NW

_TC_BLK = 4000


def _proj_body(xu_ref, xm_ref, wt_ref, b_ref, u_ref, m_ref):
    wt = wt_ref[...]
    u_ref[...] = jnp.dot(
        xu_ref[...], wt[:HIDDEN],
        preferred_element_type=jnp.float32,
        precision=jax.lax.Precision.HIGHEST,
    ) + b_ref[...]
    m_ref[...] = jnp.dot(
        xm_ref[...], wt[HIDDEN:],
        preferred_element_type=jnp.float32,
        precision=jax.lax.Precision.HIGHEST,
    )


def _project(x_user, x_movie, wt, bp):
    grid = N_NODES // _TC_BLK
    return pl.pallas_call(
        _proj_body,
        grid=(grid,),
        in_specs=[
            pl.BlockSpec((_TC_BLK, HIDDEN), lambda i: (i, 0)),
            pl.BlockSpec((_TC_BLK, HIDDEN), lambda i: (i, 0)),
            pl.BlockSpec((2 * HIDDEN, D), lambda i: (0, 0)),
            pl.BlockSpec((1, D), lambda i: (0, 0)),
        ],
        out_specs=[
            pl.BlockSpec((_TC_BLK, D), lambda i: (i, 0)),
            pl.BlockSpec((_TC_BLK, D), lambda i: (i, 0)),
        ],
        out_shape=[
            jax.ShapeDtypeStruct((N_NODES, D), jnp.float32),
            jax.ShapeDtypeStruct((N_NODES, D), jnp.float32),
        ],
    )(x_user, x_movie, wt, bp)


def _gather_add(u_tab, m_tab, idx, ncls):
    mesh = plsc.VectorSubcoreMesh(core_axis_name="c", subcore_axis_name="s")

    @functools.partial(
        pl.kernel,
        mesh=mesh,
        compiler_params=pltpu.CompilerParams(use_tc_tiling_on_sc=False),
        out_type=jax.ShapeDtypeStruct((E, ncls), jnp.float32),
        scratch_types=[
            pltpu.VMEM((2, _CH), jnp.int32),
            pltpu.VMEM((2, _CH), jnp.int32),
            pltpu.VMEM((_CH, D), jnp.float32),
            pltpu.VMEM((_CH, D), jnp.float32),
            pltpu.VMEM((_CH, D), jnp.float32),
            pltpu.VMEM((_CH, D), jnp.float32),
            pltpu.SemaphoreType.DMA,
            pltpu.SemaphoreType.DMA,
            pltpu.SemaphoreType.DMA,
            pltpu.SemaphoreType.DMA,
        ],
    )
    def k(u_hbm, m_hbm, idx_hbm, out_hbm, i0_v, i1_v, ua, ma, ub, mb,
          su_a, sm_a, su_b, sm_b):
        wid = lax.axis_index("s") * _NC + lax.axis_index("c")

        def load_and_fire(c, slot, u_rows, m_rows, su, sm):
            base = c * _CH
            pltpu.sync_copy(idx_hbm.at[0, pl.ds(base, _CH)], i0_v.at[slot])
            pltpu.sync_copy(idx_hbm.at[1, pl.ds(base, _CH)], i1_v.at[slot])
            cu = pltpu.async_copy(u_hbm.at[i0_v.at[slot]], u_rows, su)
            cm = pltpu.async_copy(m_hbm.at[i1_v.at[slot]], m_rows, sm)
            return cu, cm

        def add_and_store(c, u_rows, m_rows):
            @plsc.parallel_loop(0, _CH, step=1, unroll=8)
            def _row(i):
                u_rows[i, :] = u_rows[i, :] + m_rows[i, :]

            base = c * _CH
            pltpu.sync_copy(u_rows.at[:, :ncls], out_hbm.at[pl.ds(base, _CH)])

        def pair(j, carry):
            ca = wid + _NW * (2 * j)
            cb = wid + _NW * (2 * j + 1)
            cua, cma = load_and_fire(ca, 0, ua, ma, su_a, sm_a)
            cub, cmb = load_and_fire(cb, 1, ub, mb, su_b, sm_b)
            cua.wait()
            cma.wait()
            add_and_store(ca, ua, ma)
            cub.wait()
            cmb.wait()
            add_and_store(cb, ub, mb)
            return carry

        lax.fori_loop(0, _ROUNDS // 2, pair, 0)

        c_last = wid + _NW * (_ROUNDS - 1)
        cua, cma = load_and_fire(c_last, 0, ua, ma, su_a, sm_a)

        @pl.when(wid < _TAIL)
        def _fire_tail():
            c_tail = _ROUNDS * _NW + wid
            cub, cmb = load_and_fire(c_tail, 1, ub, mb, su_b, sm_b)

        cua.wait()
        cma.wait()
        add_and_store(c_last, ua, ma)

        @pl.when(wid < _TAIL)
        def _do_tail():
            c_tail = _ROUNDS * _NW + wid
            pltpu.make_async_copy(u_hbm.at[i0_v.at[1]], ub, su_b).wait()
            pltpu.make_async_copy(m_hbm.at[i1_v.at[1]], mb, sm_b).wait()
            add_and_store(c_tail, ub, mb)

    return k(u_tab, m_tab, idx)


def kernel(x_user, x_movie, edge_label_index, W, b):
    ncls = W.shape[0]
    idx = edge_label_index.astype(jnp.int32)
    wt = jnp.zeros((2 * HIDDEN, D), jnp.float32).at[:, :ncls].set(W.T)
    bp = jnp.zeros((1, D), jnp.float32).at[0, :ncls].set(b)
    u_tab, m_tab = _project(x_user, x_movie, wt, bp)
    return _gather_add(u_tab, m_tab, idx, ncls)


# speedup vs baseline: 5.8707x; 3.0956x over previous
import functools

import jax
import jax.numpy as jnp
from jax import lax
from jax.experimental import pallas as pl
from jax.experimental.pallas import tpu as pltpu
from jax.experimental.pallas import tpu_sc as plsc

HIDDEN = 64
N_NODES = 100000
E = 1000000
D = 8

_NC = 2
_NS = 16
_NW = _NC * _NS
_CH = 1000
_NCHUNK = E // _CH
_ROUNDS = _NCHUNK // _NW
_TAIL = _NCHUNK - _ROUNDS * _NW

_TC_BLK = 3584


def _proj_body(xu_ref, xm_ref, wt_ref, b_ref, u_ref, m_ref):
    wt = wt_ref[...]
    bias = b_ref[...]
    u = jnp.dot(
        xu_ref[...], wt[:HIDDEN],
        preferred_element_type=jnp.float32,
        precision=jax.lax.Precision.HIGHEST,
    ) + bias
    m = jnp.dot(
        xm_ref[...], wt[HIDDEN:],
        preferred_element_type=jnp.float32,
        precision=jax.lax.Precision.HIGHEST,
    )
    u_ref[...] = u
    m_ref[...] = m


def _project(x_user, x_movie, wt, bp):
    grid = -(-N_NODES // _TC_BLK)
    return pl.pallas_call(
        _proj_body,
        grid=(grid,),
        in_specs=[
            pl.BlockSpec((_TC_BLK, HIDDEN), lambda i: (i, 0)),
            pl.BlockSpec((_TC_BLK, HIDDEN), lambda i: (i, 0)),
            pl.BlockSpec((2 * HIDDEN, D), lambda i: (0, 0)),
            pl.BlockSpec((1, D), lambda i: (0, 0)),
        ],
        out_specs=[
            pl.BlockSpec((_TC_BLK, D), lambda i: (i, 0)),
            pl.BlockSpec((_TC_BLK, D), lambda i: (i, 0)),
        ],
        out_shape=[
            jax.ShapeDtypeStruct((N_NODES, D), jnp.float32),
            jax.ShapeDtypeStruct((N_NODES, D), jnp.float32),
        ],
    )(x_user, x_movie, wt, bp)


def _gather_add(u_tab, m_tab, idx):
    mesh = plsc.VectorSubcoreMesh(core_axis_name="c", subcore_axis_name="s")

    @functools.partial(
        pl.kernel,
        mesh=mesh,
        compiler_params=pltpu.CompilerParams(
            use_tc_tiling_on_sc=False, needs_layout_passes=False),
        out_type=jax.ShapeDtypeStruct((E * D,), jnp.float32),
        scratch_types=[
            pltpu.VMEM((2, _CH), jnp.int32),
            pltpu.VMEM((2, _CH), jnp.int32),
            pltpu.VMEM((_CH, D), jnp.float32),
            pltpu.VMEM((_CH, D), jnp.float32),
            pltpu.VMEM((_CH, D), jnp.float32),
            pltpu.VMEM((_CH, D), jnp.float32),
            pltpu.VMEM((_CH * D,), jnp.float32),
            pltpu.VMEM((_CH * D,), jnp.float32),
            pltpu.SemaphoreType.DMA,
            pltpu.SemaphoreType.DMA,
            pltpu.SemaphoreType.DMA,
            pltpu.SemaphoreType.DMA,
        ],
    )
    def k(u_hbm, m_hbm, idx_hbm, out_hbm, i0_v, i1_v, ua, ma, ub, mb,
          fa, fb, su_a, sm_a, su_b, sm_b):
        wid = lax.axis_index("s") * _NC + lax.axis_index("c")
        lane = lax.iota(jnp.int32, 16)
        row_off = lane >> 3
        col = lane & 7

        def load_and_fire(c, slot, u_rows, m_rows, su, sm):
            base = c * _CH
            pltpu.sync_copy(idx_hbm.at[0, pl.ds(base, _CH)], i0_v.at[slot])
            pltpu.sync_copy(idx_hbm.at[1, pl.ds(base, _CH)], i1_v.at[slot])
            cu = pltpu.async_copy(u_hbm.at[i0_v.at[slot]], u_rows, su)
            cm = pltpu.async_copy(m_hbm.at[i1_v.at[slot]], m_rows, sm)
            return cu, cm

        def add_and_store(c, u_rows, m_rows, flat):
            @plsc.parallel_loop(0, _CH // 2, step=1, unroll=8)
            def _pair_rows(j):
                r = row_off + 2 * j
                s = (plsc.load_gather(u_rows, [r, col])
                     + plsc.load_gather(m_rows, [r, col]))
                flat[pl.ds(16 * j, 16)] = s

            pltpu.sync_copy(flat, out_hbm.at[pl.ds(c * (_CH * D), _CH * D)])

        def pair(j, carry):
            ca = wid + _NW * (2 * j)
            cb = wid + _NW * (2 * j + 1)
            cua, cma = load_and_fire(ca, 0, ua, ma, su_a, sm_a)
            cub, cmb = load_and_fire(cb, 1, ub, mb, su_b, sm_b)
            cua.wait()
            cma.wait()
            add_and_store(ca, ua, ma, fa)
            cub.wait()
            cmb.wait()
            add_and_store(cb, ub, mb, fb)
            return carry

        lax.fori_loop(0, _ROUNDS // 2, pair, 0)

        c_last = wid + _NW * (_ROUNDS - 1)
        cua, cma = load_and_fire(c_last, 0, ua, ma, su_a, sm_a)

        @pl.when(wid < _TAIL)
        def _fire_tail():
            load_and_fire(_ROUNDS * _NW + wid, 1, ub, mb, su_b, sm_b)

        cua.wait()
        cma.wait()
        add_and_store(c_last, ua, ma, fa)

        @pl.when(wid < _TAIL)
        def _do_tail():
            pltpu.make_async_copy(u_hbm.at[i0_v.at[1]], ub, su_b).wait()
            pltpu.make_async_copy(m_hbm.at[i1_v.at[1]], mb, sm_b).wait()
            add_and_store(_ROUNDS * _NW + wid, ub, mb, fb)

    return k(u_tab, m_tab, idx)


def kernel(x_user, x_movie, edge_label_index, W, b):
    ncls = W.shape[0]
    idx = edge_label_index.astype(jnp.int32)
    wt = jnp.zeros((2 * HIDDEN, D), jnp.float32).at[:, :ncls].set(W.T)
    bp = jnp.zeros((1, D), jnp.float32).at[0, :ncls].set(b)
    u_tab, m_tab = _project(x_user, x_movie, wt, bp)
    out_flat = _gather_add(u_tab, m_tab, idx)
    return out_flat.reshape(E, D)[:, :ncls]


# speedup vs baseline: 6.8270x; 1.1629x over previous
import functools

import jax
import jax.numpy as jnp
from jax import lax
from jax.experimental import pallas as pl
from jax.experimental.pallas import tpu as pltpu
from jax.experimental.pallas import tpu_sc as plsc

HIDDEN = 64
N_NODES = 100000
E = 1000000
D = 8

_NC = 2
_NS = 16
_NW = _NC * _NS
_CH = 1600
_NCHUNK = E // _CH
_ROUNDS = _NCHUNK // _NW
_TAIL = _NCHUNK - _ROUNDS * _NW
_ORPC = _CH * D // 128

_KP = 16
_XW = _KP * HIDDEN
_TC_BLK = 1256


def _proj_body(xu_ref, xm_ref, wku_ref, wkm_ref, b_ref, u_ref, m_ref):
    u_ref[...] = jnp.dot(
        xu_ref[...], wku_ref[...], preferred_element_type=jnp.float32,
    ) + b_ref[...]
    m_ref[...] = jnp.dot(
        xm_ref[...], wkm_ref[...], preferred_element_type=jnp.float32,
    )


def _project(xu2, xm2, wku, wkm, bk):
    grid = -(-(N_NODES // _KP) // _TC_BLK)
    return pl.pallas_call(
        _proj_body,
        grid=(grid,),
        in_specs=[
            pl.BlockSpec((_TC_BLK, _XW), lambda i: (i, 0)),
            pl.BlockSpec((_TC_BLK, _XW), lambda i: (i, 0)),
            pl.BlockSpec((_XW, 128), lambda i: (0, 0)),
            pl.BlockSpec((_XW, 128), lambda i: (0, 0)),
            pl.BlockSpec((1, 128), lambda i: (0, 0)),
        ],
        out_specs=[
            pl.BlockSpec((_TC_BLK, 128), lambda i: (i, 0)),
            pl.BlockSpec((_TC_BLK, 128), lambda i: (i, 0)),
        ],
        out_shape=[
            jax.ShapeDtypeStruct((N_NODES // _KP, 128), jnp.float32),
            jax.ShapeDtypeStruct((N_NODES // _KP, 128), jnp.float32),
        ],
    )(xu2, xm2, wku, wkm, bk)


def _gather_add(u_tab, m_tab, idx):
    mesh = plsc.VectorSubcoreMesh(core_axis_name="c", subcore_axis_name="s")

    @functools.partial(
        pl.kernel,
        mesh=mesh,
        compiler_params=pltpu.CompilerParams(
            use_tc_tiling_on_sc=False, needs_layout_passes=False),
        out_type=jax.ShapeDtypeStruct((E * D // 128, 128), jnp.float32),
        scratch_types=[
            pltpu.VMEM((2, _CH), jnp.int32),
            pltpu.VMEM((2, _CH), jnp.int32),
            pltpu.VMEM((_CH, D), jnp.float32),
            pltpu.VMEM((_CH, D), jnp.float32),
            pltpu.VMEM((_CH, D), jnp.float32),
            pltpu.VMEM((_CH, D), jnp.float32),
            pltpu.VMEM((_ORPC, 128), jnp.float32),
            pltpu.VMEM((_ORPC, 128), jnp.float32),
            pltpu.SemaphoreType.DMA,
            pltpu.SemaphoreType.DMA,
            pltpu.SemaphoreType.DMA,
            pltpu.SemaphoreType.DMA,
        ],
    )
    def k(u_hbm, m_hbm, idx_hbm, out_hbm, i0_v, i1_v, ua, ma, ub, mb,
          fa, fb, su_a, sm_a, su_b, sm_b):
        wid = lax.axis_index("s") * _NC + lax.axis_index("c")
        lane = lax.iota(jnp.int32, 16)
        row_off = lane >> 3
        col = lane & 7

        def load_and_fire(c, slot, u_rows, m_rows, su, sm):
            base = c * _CH
            pltpu.sync_copy(idx_hbm.at[0, pl.ds(base, _CH)], i0_v.at[slot])
            pltpu.sync_copy(idx_hbm.at[1, pl.ds(base, _CH)], i1_v.at[slot])
            cu = pltpu.async_copy(u_hbm.at[i0_v.at[slot]], u_rows, su)
            cm = pltpu.async_copy(m_hbm.at[i1_v.at[slot]], m_rows, sm)
            return cu, cm

        def add_and_store(c, u_rows, m_rows, flat):
            @plsc.parallel_loop(0, _CH // 2, step=1, unroll=8)
            def _pair_rows(j):
                r = row_off + 2 * j
                s = (plsc.load_gather(u_rows, [r, col])
                     + plsc.load_gather(m_rows, [r, col]))
                flat[j >> 3, pl.ds(16 * (j & 7), 16)] = s

            pltpu.sync_copy(flat, out_hbm.at[pl.ds(c * _ORPC, _ORPC)])

        def pair(j, carry):
            ca = wid + _NW * (2 * j)
            cb = wid + _NW * (2 * j + 1)
            cua, cma = load_and_fire(ca, 0, ua, ma, su_a, sm_a)
            cub, cmb = load_and_fire(cb, 1, ub, mb, su_b, sm_b)
            cua.wait()
            cma.wait()
            add_and_store(ca, ua, ma, fa)
            cub.wait()
            cmb.wait()
            add_and_store(cb, ub, mb, fb)
            return carry

        lax.fori_loop(0, _ROUNDS // 2, pair, 0)

        c_last = wid + _NW * (_ROUNDS - 1)
        cua, cma = load_and_fire(c_last, 0, ua, ma, su_a, sm_a)

        @pl.when(wid < _TAIL)
        def _fire_tail():
            load_and_fire(_ROUNDS * _NW + wid, 1, ub, mb, su_b, sm_b)

        cua.wait()
        cma.wait()
        add_and_store(c_last, ua, ma, fa)

        @pl.when(wid < _TAIL)
        def _do_tail():
            pltpu.make_async_copy(u_hbm.at[i0_v.at[1]], ub, su_b).wait()
            pltpu.make_async_copy(m_hbm.at[i1_v.at[1]], mb, sm_b).wait()
            add_and_store(_ROUNDS * _NW + wid, ub, mb, fb)

    return k(u_tab, m_tab, idx)


def kernel(x_user, x_movie, edge_label_index, W, b):
    ncls = W.shape[0]
    idx = edge_label_index.astype(jnp.int32)
    wtu = jnp.zeros((HIDDEN, D), jnp.float32).at[:, :ncls].set(W[:, :HIDDEN].T)
    wtm = jnp.zeros((HIDDEN, D), jnp.float32).at[:, :ncls].set(W[:, HIDDEN:].T)
    eye = jnp.eye(_KP, dtype=jnp.float32)
    wku = jnp.kron(eye, wtu)
    wkm = jnp.kron(eye, wtm)
    bp = jnp.zeros((D,), jnp.float32).at[:ncls].set(b)
    bk = jnp.tile(bp, _KP).reshape(1, _KP * D)
    xu2 = x_user.reshape(N_NODES // _KP, _XW)
    xm2 = x_movie.reshape(N_NODES // _KP, _XW)
    u6, m6 = _project(xu2, xm2, wku, wkm, bk)
    u_tab = u6.reshape(N_NODES, D)
    m_tab = m6.reshape(N_NODES, D)
    g2 = _gather_add(u_tab, m_tab, idx)
    return g2.reshape(E, D)[:, :ncls]


# speedup vs baseline: 15.6908x; 2.2983x over previous
import functools

import jax
import jax.numpy as jnp
from jax import lax
from jax.experimental import pallas as pl
from jax.experimental.pallas import tpu as pltpu
from jax.experimental.pallas import tpu_sc as plsc

HIDDEN = 64
N_NODES = 100000
E = 1000000
D = 8

_NC = 2
_NS = 16
_NW = _NC * _NS
_CH = 1280
_NCHUNK = (E - 320) // _CH
_ROUNDS = _NCHUNK // _NW
_TAIL = _NCHUNK - _ROUNDS * _NW
_TPC = _CH // 128
_RG = 320
_RGB = 384
_TROWS = E * D // (D * 128) + 1

_KP = 16
_XW = _KP * HIDDEN
_TC_BLK = 1256


def _proj_body(xu_ref, xm_ref, wku_ref, wkm_ref, b_ref, u_ref, m_ref):
    u_ref[...] = jnp.dot(
        xu_ref[...], wku_ref[...], preferred_element_type=jnp.float32,
    ) + b_ref[...]
    m_ref[...] = jnp.dot(
        xm_ref[...], wkm_ref[...], preferred_element_type=jnp.float32,
    )


def _project(xu2, xm2, wku, wkm, bk):
    grid = -(-(N_NODES // _KP) // _TC_BLK)
    return pl.pallas_call(
        _proj_body,
        grid=(grid,),
        in_specs=[
            pl.BlockSpec((_TC_BLK, _XW), lambda i: (i, 0)),
            pl.BlockSpec((_TC_BLK, _XW), lambda i: (i, 0)),
            pl.BlockSpec((_XW, 128), lambda i: (0, 0)),
            pl.BlockSpec((_XW, 128), lambda i: (0, 0)),
            pl.BlockSpec((1, 128), lambda i: (0, 0)),
        ],
        out_specs=[
            pl.BlockSpec((_TC_BLK, 128), lambda i: (i, 0)),
            pl.BlockSpec((_TC_BLK, 128), lambda i: (i, 0)),
        ],
        out_shape=[
            jax.ShapeDtypeStruct((N_NODES // _KP, 128), jnp.float32),
            jax.ShapeDtypeStruct((N_NODES // _KP, 128), jnp.float32),
        ],
    )(xu2, xm2, wku, wkm, bk)


def _gather_add(u_tab, m_tab, idx):
    mesh = plsc.VectorSubcoreMesh(core_axis_name="c", subcore_axis_name="s")

    @functools.partial(
        pl.kernel,
        mesh=mesh,
        compiler_params=pltpu.CompilerParams(
            use_tc_tiling_on_sc=False, needs_layout_passes=False),
        out_type=jax.ShapeDtypeStruct((_TROWS, D, 128), jnp.float32),
        scratch_types=[
            pltpu.VMEM((2, _CH), jnp.int32),
            pltpu.VMEM((2, _CH), jnp.int32),
            pltpu.VMEM((_CH, D), jnp.float32),
            pltpu.VMEM((_CH, D), jnp.float32),
            pltpu.VMEM((_CH, D), jnp.float32),
            pltpu.VMEM((_CH, D), jnp.float32),
            pltpu.VMEM((_TPC, D, 128), jnp.float32),
            pltpu.VMEM((_TPC, D, 128), jnp.float32),
            pltpu.VMEM((_RG,), jnp.int32),
            pltpu.VMEM((_RG,), jnp.int32),
            pltpu.VMEM((_RGB, D), jnp.float32),
            pltpu.VMEM((_RGB, D), jnp.float32),
            pltpu.SemaphoreType.DMA,
            pltpu.SemaphoreType.DMA,
            pltpu.SemaphoreType.DMA,
            pltpu.SemaphoreType.DMA,
        ],
    )
    def k(u_hbm, m_hbm, idx_hbm, out_hbm, i0_v, i1_v, ua, ma, ub, mb,
          fa, fb, i0r, i1r, ur, mr, su_a, sm_a, su_b, sm_b):
        wid = lax.axis_index("s") * _NC + lax.axis_index("c")
        lane = lax.iota(jnp.int32, 16)

        def load_and_fire(c, slot, u_rows, m_rows, su, sm):
            base = c * _CH
            pltpu.sync_copy(idx_hbm.at[0, pl.ds(base, _CH)], i0_v.at[slot])
            pltpu.sync_copy(idx_hbm.at[1, pl.ds(base, _CH)], i1_v.at[slot])
            cu = pltpu.async_copy(u_hbm.at[i0_v.at[slot]], u_rows, su)
            cm = pltpu.async_copy(m_hbm.at[i1_v.at[slot]], m_rows, sm)
            return cu, cm

        def add_tiles(niter, u_rows, m_rows, flat):
            @plsc.parallel_loop(0, niter, step=1, unroll=8)
            def _vec(j):
                tt = j >> 6
                c = (j >> 3) & 7
                lg = j & 7
                r = tt * 128 + lg * 16 + lane
                cv = jnp.full((16,), c, jnp.int32)
                sv = (plsc.load_gather(u_rows, [r, cv])
                      + plsc.load_gather(m_rows, [r, cv]))
                flat[tt, c, pl.ds(lg * 16, 16)] = sv

        def add_and_store(c, u_rows, m_rows, flat):
            add_tiles(_CH * D // 16, u_rows, m_rows, flat)
            pltpu.sync_copy(flat, out_hbm.at[pl.ds(c * _TPC, _TPC)])

        def pair(j, carry):
            ca = wid + _NW * (2 * j)
            cb = wid + _NW * (2 * j + 1)
            cua, cma = load_and_fire(ca, 0, ua, ma, su_a, sm_a)
            cub, cmb = load_and_fire(cb, 1, ub, mb, su_b, sm_b)
            cua.wait()
            cma.wait()
            add_and_store(ca, ua, ma, fa)
            cub.wait()
            cmb.wait()
            add_and_store(cb, ub, mb, fb)
            return carry

        lax.fori_loop(0, _ROUNDS // 2, pair, 0)

        @pl.when(wid < _TAIL)
        def _fire_tail():
            load_and_fire(_ROUNDS * _NW + wid, 1, ub, mb, su_b, sm_b)

        @pl.when(wid == _NW - 1)
        def _fire_rag():
            pltpu.sync_copy(idx_hbm.at[0, pl.ds(E - _RG, _RG)], i0r)
            pltpu.sync_copy(idx_hbm.at[1, pl.ds(E - _RG, _RG)], i1r)
            pltpu.async_copy(u_hbm.at[i0r], ur.at[pl.ds(0, _RG)], su_a)
            pltpu.async_copy(m_hbm.at[i1r], mr.at[pl.ds(0, _RG)], sm_a)

        @pl.when(wid < _TAIL)
        def _do_tail():
            pltpu.make_async_copy(u_hbm.at[i0_v.at[1]], ub, su_b).wait()
            pltpu.make_async_copy(m_hbm.at[i1_v.at[1]], mb, sm_b).wait()
            add_and_store(_ROUNDS * _NW + wid, ub, mb, fb)

        @pl.when(wid == _NW - 1)
        def _do_rag():
            pltpu.make_async_copy(
                u_hbm.at[i0r], ur.at[pl.ds(0, _RG)], su_a).wait()
            pltpu.make_async_copy(
                m_hbm.at[i1r], mr.at[pl.ds(0, _RG)], sm_a).wait()
            add_tiles(3 * D * 8, ur, mr, fa)
            pltpu.sync_copy(fa.at[pl.ds(0, 3)],
                            out_hbm.at[pl.ds(_TROWS - 3, 3)])

    return k(u_tab, m_tab, idx)


_FB = 632


def _declass_body(g_ref, o_ref):
    t = jnp.transpose(g_ref[...], (1, 0, 2))
    o_ref[...] = t.reshape(D, _FB * 128)[:7, :]


def _declassify(g3, ncls):
    return pl.pallas_call(
        _declass_body,
        grid=(-(-_TROWS // _FB),),
        in_specs=[pl.BlockSpec((_FB, D, 128), lambda i: (i, 0, 0))],
        out_specs=pl.BlockSpec((ncls, _FB * 128), lambda i: (0, i)),
        out_shape=jax.ShapeDtypeStruct((ncls, E), jnp.float32),
    )(g3)


def kernel(x_user, x_movie, edge_label_index, W, b):
    ncls = W.shape[0]
    idx = edge_label_index.astype(jnp.int32)
    wtu = jnp.zeros((HIDDEN, D), jnp.float32).at[:, :ncls].set(W[:, :HIDDEN].T)
    wtm = jnp.zeros((HIDDEN, D), jnp.float32).at[:, :ncls].set(W[:, HIDDEN:].T)
    eye = jnp.eye(_KP, dtype=jnp.float32)
    wku = jnp.kron(eye, wtu)
    wkm = jnp.kron(eye, wtm)
    bp = jnp.zeros((D,), jnp.float32).at[:ncls].set(b)
    bk = jnp.tile(bp, _KP).reshape(1, _KP * D)
    xu2 = x_user.reshape(N_NODES // _KP, _XW)
    xm2 = x_movie.reshape(N_NODES // _KP, _XW)
    u6, m6 = _project(xu2, xm2, wku, wkm, bk)
    u_tab = u6.reshape(N_NODES, D)
    m_tab = m6.reshape(N_NODES, D)
    g3 = _gather_add(u_tab, m_tab, idx)
    out_t = _declassify(g3, ncls)
    return out_t.T


# speedup vs baseline: 15.8348x; 1.0092x over previous
import functools

import jax
import jax.numpy as jnp
from jax import lax
from jax.experimental import pallas as pl
from jax.experimental.pallas import tpu as pltpu
from jax.experimental.pallas import tpu_sc as plsc

HIDDEN = 64
N_NODES = 100000
E = 1000000
D = 8

_NC = 2
_NS = 16
_NW = _NC * _NS
_CH = 1280
_NCHUNK = (E - 320) // _CH
_ROUNDS = _NCHUNK // _NW
_TAIL = _NCHUNK - _ROUNDS * _NW
_TPC = _CH // 128
_RG = 320
_RGB = 384
_TROWS = E * D // (D * 128) + 1

_KP = 16
_XW = _KP * HIDDEN
_TC_BLK = 1256


def _proj_body(xu_ref, xm_ref, wku_ref, wkm_ref, b_ref, u_ref, m_ref):
    xu = xu_ref[...].reshape(_TC_BLK, _XW)
    xm = xm_ref[...].reshape(_TC_BLK, _XW)
    u_ref[...] = jnp.dot(
        xu, wku_ref[...], preferred_element_type=jnp.float32,
    ) + b_ref[...]
    m_ref[...] = jnp.dot(
        xm, wkm_ref[...], preferred_element_type=jnp.float32,
    )


def _project(xu2, xm2, wku, wkm, bk):
    grid = -(-(N_NODES // _KP) // _TC_BLK)
    return pl.pallas_call(
        _proj_body,
        grid=(grid,),
        in_specs=[
            pl.BlockSpec((8 * _TC_BLK, 128), lambda i: (i, 0)),
            pl.BlockSpec((8 * _TC_BLK, 128), lambda i: (i, 0)),
            pl.BlockSpec((_XW, 128), lambda i: (0, 0)),
            pl.BlockSpec((_XW, 128), lambda i: (0, 0)),
            pl.BlockSpec((1, 128), lambda i: (0, 0)),
        ],
        out_specs=[
            pl.BlockSpec((_TC_BLK, 128), lambda i: (i, 0)),
            pl.BlockSpec((_TC_BLK, 128), lambda i: (i, 0)),
        ],
        out_shape=[
            jax.ShapeDtypeStruct((N_NODES // _KP, 128), jnp.float32),
            jax.ShapeDtypeStruct((N_NODES // _KP, 128), jnp.float32),
        ],
    )(xu2, xm2, wku, wkm, bk)


def _gather_add(u_tab, m_tab, idx):
    mesh = plsc.VectorSubcoreMesh(core_axis_name="c", subcore_axis_name="s")

    @functools.partial(
        pl.kernel,
        mesh=mesh,
        compiler_params=pltpu.CompilerParams(
            use_tc_tiling_on_sc=False, needs_layout_passes=False),
        out_type=jax.ShapeDtypeStruct((_TROWS, D, 128), jnp.float32),
        scratch_types=[
            pltpu.VMEM((2, _CH), jnp.int32),
            pltpu.VMEM((2, _CH), jnp.int32),
            pltpu.VMEM((_CH, D), jnp.float32),
            pltpu.VMEM((_CH, D), jnp.float32),
            pltpu.VMEM((_CH, D), jnp.float32),
            pltpu.VMEM((_CH, D), jnp.float32),
            pltpu.VMEM((_TPC, D, 128), jnp.float32),
            pltpu.VMEM((_TPC, D, 128), jnp.float32),
            pltpu.VMEM((_RG,), jnp.int32),
            pltpu.VMEM((_RG,), jnp.int32),
            pltpu.VMEM((_RGB, D), jnp.float32),
            pltpu.VMEM((_RGB, D), jnp.float32),
            pltpu.SemaphoreType.DMA,
            pltpu.SemaphoreType.DMA,
            pltpu.SemaphoreType.DMA,
            pltpu.SemaphoreType.DMA,
        ],
    )
    def k(u_hbm, m_hbm, idx_hbm, out_hbm, i0_v, i1_v, ua, ma, ub, mb,
          fa, fb, i0r, i1r, ur, mr, su_a, sm_a, su_b, sm_b):
        wid = lax.axis_index("s") * _NC + lax.axis_index("c")
        lane = lax.iota(jnp.int32, 16)

        def load_and_fire(c, slot, u_rows, m_rows, su, sm):
            base = c * _CH
            pltpu.sync_copy(idx_hbm.at[0, pl.ds(base, _CH)], i0_v.at[slot])
            pltpu.sync_copy(idx_hbm.at[1, pl.ds(base, _CH)], i1_v.at[slot])
            cu = pltpu.async_copy(u_hbm.at[i0_v.at[slot]], u_rows, su)
            cm = pltpu.async_copy(m_hbm.at[i1_v.at[slot]], m_rows, sm)
            return cu, cm

        def add_tiles(niter, u_rows, m_rows, flat):
            @plsc.parallel_loop(0, niter, step=1, unroll=8)
            def _vec(j):
                tt = j >> 6
                c = (j >> 3) & 7
                lg = j & 7
                r = tt * 128 + lg * 16 + lane
                cv = jnp.full((16,), c, jnp.int32)
                sv = (plsc.load_gather(u_rows, [r, cv])
                      + plsc.load_gather(m_rows, [r, cv]))
                flat[tt, c, pl.ds(lg * 16, 16)] = sv

        def add_and_store(c, u_rows, m_rows, flat):
            add_tiles(_CH * D // 16, u_rows, m_rows, flat)
            pltpu.sync_copy(flat, out_hbm.at[pl.ds(c * _TPC, _TPC)])

        def pair(j, carry):
            ca = wid + _NW * (2 * j)
            cb = wid + _NW * (2 * j + 1)
            cua, cma = load_and_fire(ca, 0, ua, ma, su_a, sm_a)
            cub, cmb = load_and_fire(cb, 1, ub, mb, su_b, sm_b)
            cua.wait()
            cma.wait()
            add_and_store(ca, ua, ma, fa)
            cub.wait()
            cmb.wait()
            add_and_store(cb, ub, mb, fb)
            return carry

        lax.fori_loop(0, _ROUNDS // 2, pair, 0)

        @pl.when(wid < _TAIL)
        def _fire_tail():
            load_and_fire(_ROUNDS * _NW + wid, 1, ub, mb, su_b, sm_b)

        @pl.when(wid == _NW - 1)
        def _fire_rag():
            pltpu.sync_copy(idx_hbm.at[0, pl.ds(E - _RG, _RG)], i0r)
            pltpu.sync_copy(idx_hbm.at[1, pl.ds(E - _RG, _RG)], i1r)
            pltpu.async_copy(u_hbm.at[i0r], ur.at[pl.ds(0, _RG)], su_a)
            pltpu.async_copy(m_hbm.at[i1r], mr.at[pl.ds(0, _RG)], sm_a)

        @pl.when(wid < _TAIL)
        def _do_tail():
            pltpu.make_async_copy(u_hbm.at[i0_v.at[1]], ub, su_b).wait()
            pltpu.make_async_copy(m_hbm.at[i1_v.at[1]], mb, sm_b).wait()
            add_and_store(_ROUNDS * _NW + wid, ub, mb, fb)

        @pl.when(wid == _NW - 1)
        def _do_rag():
            pltpu.make_async_copy(
                u_hbm.at[i0r], ur.at[pl.ds(0, _RG)], su_a).wait()
            pltpu.make_async_copy(
                m_hbm.at[i1r], mr.at[pl.ds(0, _RG)], sm_a).wait()
            add_tiles(3 * D * 8, ur, mr, fa)
            pltpu.sync_copy(fa.at[pl.ds(0, 3)],
                            out_hbm.at[pl.ds(_TROWS - 3, 3)])

    return k(u_tab, m_tab, idx)


_FB = 632


def _declass_body(g_ref, o_ref):
    t = jnp.transpose(g_ref[...], (1, 0, 2))
    o_ref[...] = t.reshape(D, _FB * 128)[:7, :]


def _declassify(g3, ncls):
    return pl.pallas_call(
        _declass_body,
        grid=(-(-_TROWS // _FB),),
        in_specs=[pl.BlockSpec((_FB, D, 128), lambda i: (i, 0, 0))],
        out_specs=pl.BlockSpec((ncls, _FB * 128), lambda i: (0, i)),
        out_shape=jax.ShapeDtypeStruct((ncls, E), jnp.float32),
    )(g3)


def kernel(x_user, x_movie, edge_label_index, W, b):
    ncls = W.shape[0]
    idx = edge_label_index.astype(jnp.int32)
    wtu = jnp.zeros((HIDDEN, D), jnp.float32).at[:, :ncls].set(W[:, :HIDDEN].T)
    wtm = jnp.zeros((HIDDEN, D), jnp.float32).at[:, :ncls].set(W[:, HIDDEN:].T)
    eye = jnp.eye(_KP, dtype=jnp.float32)
    wku = jnp.kron(eye, wtu)
    wkm = jnp.kron(eye, wtm)
    bp = jnp.zeros((D,), jnp.float32).at[:ncls].set(b)
    bk = jnp.tile(bp, _KP).reshape(1, _KP * D)
    xu2 = x_user.reshape(N_NODES * HIDDEN // 128, 128)
    xm2 = x_movie.reshape(N_NODES * HIDDEN // 128, 128)
    u6, m6 = _project(xu2, xm2, wku, wkm, bk)
    u_tab = u6.reshape(N_NODES, D)
    m_tab = m6.reshape(N_NODES, D)
    g3 = _gather_add(u_tab, m_tab, idx)
    out_t = _declassify(g3, ncls)
    return out_t.T
